# baseline (speedup 1.0000x reference)
import functools

import jax
import jax.numpy as jnp
from jax import lax
from jax.experimental import pallas as pl
from jax.experimental.pallas import tpu as pltpu
from jax.experimental.pallas import tpu_sc as plsc

N_EDGES_K = 6400000
N_FRAMES_K = 16
NC = 2
NS = 16
L = 16
NW = NC * NS
PER_W = N_EDGES_K // NW
CHUNK = 20000
N_CHUNKS = PER_W // CHUNK


def _ev_body(rij_hbm, idx_hbm, cum_hbm, out_hbm, rij_v, idx_v, cum_v, scr_v):
    wid = lax.axis_index("c") * NS + lax.axis_index("s")
    base = wid * PER_W

    pltpu.sync_copy(cum_hbm, cum_v)
    cum_vec = cum_v[pl.ds(0, N_FRAMES_K)]
    cums = [cum_vec[f] for f in range(N_FRAMES_K - 1)]

    lane = lax.iota(jnp.int32, L)
    lane3 = lane * 3

    def chunk_body(ch, accs):
        start = base + ch * CHUNK
        pltpu.sync_copy(rij_hbm.at[pl.ds(start * 3, CHUNK * 3)], rij_v)
        pltpu.sync_copy(idx_hbm.at[pl.ds(start, CHUNK)], idx_v)

        def vec_body(i, accs):
            b = i * (3 * L) + lane3
            x = plsc.load_gather(rij_v, [b])
            y = plsc.load_gather(rij_v, [b + 1])
            z = plsc.load_gather(rij_v, [b + 2])
            idxv = idx_v[pl.ds(i * L, L)]
            d2 = x * x + y * y + z * z
            r = 1.0 / d2
            e = (r * r) * r
            new = [accs[f] + jnp.where(idxv < cums[f], e, 0.0)
                   for f in range(N_FRAMES_K - 1)]
            new.append(accs[N_FRAMES_K - 1] + e)
            return tuple(new)

        return lax.fori_loop(0, CHUNK // L, vec_body, accs, unroll=2)

    zero = jnp.zeros((L,), jnp.float32)
    accs = lax.fori_loop(0, N_CHUNKS, chunk_body,
                         tuple(zero for _ in range(N_FRAMES_K)))

    for f in range(N_FRAMES_K):
        scr_v[pl.ds(f * L, L)] = accs[f]
    col_idx = lane * L
    total = plsc.load_gather(scr_v, [col_idx])
    for c in range(1, L):
        total = total + plsc.load_gather(scr_v, [col_idx + c])
    scr_v[pl.ds(0, L)] = total
    pltpu.sync_copy(scr_v.at[pl.ds(0, L)], out_hbm.at[pl.ds(wid * N_FRAMES_K, N_FRAMES_K)])


@functools.partial(jax.jit, static_argnames=())
def kernel(Rij, idx_i, n_atoms):
    cum = jnp.cumsum(n_atoms, dtype=jnp.int32)
    rij_flat = Rij.reshape(-1)

    mesh = plsc.VectorSubcoreMesh(core_axis_name="c", subcore_axis_name="s")
    run = pl.kernel(
        _ev_body,
        out_type=jax.ShapeDtypeStruct((NW * N_FRAMES_K,), jnp.float32),
        mesh=mesh,
        scratch_types=[
            pltpu.VMEM((CHUNK * 3,), jnp.float32),
            pltpu.VMEM((CHUNK,), jnp.int32),
            pltpu.VMEM((N_FRAMES_K,), jnp.int32),
            pltpu.VMEM((N_FRAMES_K * L,), jnp.float32),
        ],
        compiler_params=pltpu.CompilerParams(needs_layout_passes=False),
    )
    rows = run(rij_flat, idx_i, cum).reshape(NW, N_FRAMES_K)
    prefix = rows.sum(axis=0)
    energy = prefix - jnp.concatenate([jnp.zeros((1,), jnp.float32), prefix[:-1]])
    return energy * 0.5
